# baseline (speedup 1.0000x reference)
import functools

import jax
import jax.numpy as jnp
from jax import lax
from jax.experimental import pallas as pl
from jax.experimental.pallas import tpu as pltpu
from jax.experimental.pallas import tpu_sc as plsc

NC, NS, L = 2, 16, 16
NW = NC * NS

B, S, D = 4, 2048, 768
N = B * S
PER_W = N // NW
W_PER_B = S // PER_W
CHUNK = 16
NCHUNK = PER_W // CHUNK
NBUF = 4

_mesh = plsc.VectorSubcoreMesh(core_axis_name="c", subcore_axis_name="s")

_row_buf = [pltpu.VMEM((CHUNK, D), jnp.float32) for _ in range(2 * NBUF)]
_sems = [pltpu.SemaphoreType.DMA for _ in range(3 * NBUF)]


@functools.partial(
    pl.kernel,
    out_type=jax.ShapeDtypeStruct((N, D), jnp.float32),
    mesh=_mesh,
    scratch_types=[
        pltpu.VMEM((PER_W,), jnp.int32),
        pltpu.VMEM((PER_W,), jnp.int32),
    ] + _row_buf + _sems,
)
def _embed_sum(ids_hbm, pos_hbm, tok_tbl, pos_tbl, out_hbm,
               ids_v, pos_v, *bufs_and_sems):
    tok_rows = bufs_and_sems[0:NBUF]
    pos_rows = bufs_and_sems[NBUF:2 * NBUF]
    sem_t = bufs_and_sems[2 * NBUF:3 * NBUF]
    sem_p = bufs_and_sems[3 * NBUF:4 * NBUF]
    sem_s = bufs_and_sems[4 * NBUF:5 * NBUF]

    wid = lax.axis_index("s") * NC + lax.axis_index("c")
    base = wid * PER_W
    brow = wid // W_PER_B
    bcol = (wid % W_PER_B) * PER_W

    ci = pltpu.async_copy(ids_hbm.at[brow, pl.ds(bcol, PER_W)], ids_v, sem_t[0])
    cj = pltpu.async_copy(pos_hbm.at[brow, pl.ds(bcol, PER_W)], pos_v, sem_p[0])

    def issue_tok(c):
        b = c % NBUF
        return pltpu.async_copy(tok_tbl.at[ids_v.at[pl.ds(c * CHUNK, CHUNK)]],
                                tok_rows[b], sem_t[b])

    def issue_pos(c):
        b = c % NBUF
        return pltpu.async_copy(pos_tbl.at[pos_v.at[pl.ds(c * CHUNK, CHUNK)]],
                                pos_rows[b], sem_p[b])

    def issue(c):
        return issue_tok(c), issue_pos(c)

    pending = [None] * NBUF
    stores = [None] * NBUF
    ci.wait()
    first_tok = [issue_tok(c) for c in range(NBUF - 1)]
    cj.wait()
    for c in range(NBUF - 1):
        pending[c] = (first_tok[c], issue_pos(c))
    for c in range(NCHUNK):
        b = c % NBUF
        nxt = c + NBUF - 1
        if nxt < NCHUNK:
            nb = nxt % NBUF
            if stores[nb] is not None:
                with jax.named_scope("swait"):
                    stores[nb].wait()
            pending[nb] = issue(nxt)
        with jax.named_scope("gwait"):
            ct, cp = pending[b]
            ct.wait()
            cp.wait()

        tr, pr = tok_rows[b], pos_rows[b]

        def add_row(r, carry, tr=tr, pr=pr):
            @plsc.parallel_loop(0, D, step=L, unroll=6)
            def add_vec(j, tr=tr, pr=pr, r=r):
                sl = pl.ds(j, L)
                tr[r, sl] = tr[r, sl] + pr[r, sl]
            return carry

        with jax.named_scope("add"):
            lax.fori_loop(0, CHUNK, add_row, None)
        stores[b] = pltpu.async_copy(
            tr, out_hbm.at[pl.ds(base + c * CHUNK, CHUNK)], sem_s[b])
    for st in stores:
        if st is not None:
            st.wait()


@jax.jit
def kernel(input_ids, seq_pos, embed_tokens, seq_pos_embedding):
    ids = input_ids.astype(jnp.int32)
    pos = seq_pos.astype(jnp.int32)
    out = _embed_sum(ids, pos, embed_tokens, seq_pos_embedding)
    return out.reshape(B, S, D)


# speedup vs baseline: 1.0075x; 1.0075x over previous
import functools

import jax
import jax.numpy as jnp
from jax import lax
from jax.experimental import pallas as pl
from jax.experimental.pallas import tpu as pltpu
from jax.experimental.pallas import tpu_sc as plsc

NC, NS, L = 2, 16, 16
NW = NC * NS

B, S, D = 4, 2048, 768
N = B * S
PER_W = N // NW
W_PER_B = S // PER_W
CHUNK = 16
NCHUNK = PER_W // CHUNK
NBUF = 4

_mesh = plsc.VectorSubcoreMesh(core_axis_name="c", subcore_axis_name="s")

_row_buf = [pltpu.VMEM((CHUNK, D), jnp.float32) for _ in range(2 * NBUF)]
_sems = [pltpu.SemaphoreType.DMA for _ in range(3 * NBUF)]


@functools.partial(
    pl.kernel,
    out_type=jax.ShapeDtypeStruct((N, D), jnp.float32),
    mesh=_mesh,
    scratch_types=[
        pltpu.VMEM((PER_W,), jnp.int32),
        pltpu.VMEM((PER_W,), jnp.int32),
    ] + _row_buf + _sems,
)
def _embed_sum(ids_hbm, pos_hbm, tok_tbl, pos_tbl, out_hbm,
               ids_v, pos_v, *bufs_and_sems):
    tok_rows = bufs_and_sems[0:NBUF]
    pos_rows = bufs_and_sems[NBUF:2 * NBUF]
    sem_t = bufs_and_sems[2 * NBUF:3 * NBUF]
    sem_p = bufs_and_sems[3 * NBUF:4 * NBUF]
    sem_s = bufs_and_sems[4 * NBUF:5 * NBUF]

    wid = lax.axis_index("s") * NC + lax.axis_index("c")
    base = wid * PER_W
    brow = wid // W_PER_B
    bcol = (wid % W_PER_B) * PER_W

    ci = pltpu.async_copy(ids_hbm.at[brow, pl.ds(bcol, PER_W)], ids_v, sem_t[0])
    cj = pltpu.async_copy(pos_hbm.at[brow, pl.ds(bcol, PER_W)], pos_v, sem_p[0])

    def issue_tok(c):
        b = c % NBUF
        return pltpu.async_copy(tok_tbl.at[ids_v.at[pl.ds(c * CHUNK, CHUNK)]],
                                tok_rows[b], sem_t[b])

    def issue_pos(c):
        b = c % NBUF
        return pltpu.async_copy(pos_tbl.at[pos_v.at[pl.ds(c * CHUNK, CHUNK)]],
                                pos_rows[b], sem_p[b])

    def issue(c):
        return issue_tok(c), issue_pos(c)

    pending = [None] * NBUF
    stores = [None] * NBUF
    ci.wait()
    first_tok = [issue_tok(c) for c in range(NBUF - 1)]
    cj.wait()
    for c in range(NBUF - 1):
        pending[c] = (first_tok[c], issue_pos(c))
    for c in range(NCHUNK):
        b = c % NBUF
        nxt = c + NBUF - 1
        if nxt < NCHUNK:
            nb = nxt % NBUF
            if stores[nb] is not None:
                stores[nb].wait()
            pending[nb] = issue(nxt)
        ct, cp = pending[b]
        ct.wait()
        cp.wait()

        tr, pr = tok_rows[b], pos_rows[b]

        def add_row(r, carry, tr=tr, pr=pr):
            @plsc.parallel_loop(0, D, step=L, unroll=6)
            def add_vec(j, tr=tr, pr=pr, r=r):
                sl = pl.ds(j, L)
                tr[r, sl] = tr[r, sl] + pr[r, sl]
            return carry

        lax.fori_loop(0, CHUNK, add_row, None)
        stores[b] = pltpu.async_copy(
            tr, out_hbm.at[pl.ds(base + c * CHUNK, CHUNK)], sem_s[b])
    for st in stores:
        if st is not None:
            st.wait()


@jax.jit
def kernel(input_ids, seq_pos, embed_tokens, seq_pos_embedding):
    ids = input_ids.astype(jnp.int32)
    pos = seq_pos.astype(jnp.int32)
    out = _embed_sum(ids, pos, embed_tokens, seq_pos_embedding)
    return out.reshape(B, S, D)
